# baseline (speedup 1.0000x reference)
import jax
import jax.numpy as jnp
from jax import lax
from jax.experimental import pallas as pl
from jax.experimental.pallas import tpu as pltpu
from jax.experimental.pallas import tpu_sc as plsc

N = 10000
DEG = 32
D = 128
SLOPE = 0.01

BT = 512
NTC = 7168
NSC = N - NTC

NB = 8
NBLK = NSC // NB
NW = 32
MAXB = -(-NBLK // NW)
LANES = 16
NJ = D // LANES
RING = 3
FB = NB * DEG * D


def _tc_body(sa_ref, attn_ref, feat_ref, out_ref):
    a = attn_ref[...] + sa_ref[...]
    l = jnp.maximum(a, a * SLOPE)
    m = jnp.max(l, axis=1, keepdims=True)
    e = jnp.exp(l - m)
    w = e / jnp.sum(e, axis=1, keepdims=True)
    out_ref[...] = jnp.sum(w[:, :, None] * feat_ref[...], axis=1)


def _tc_call(self_a, attn, feat):
    return pl.pallas_call(
        _tc_body,
        grid=(NTC // BT,),
        in_specs=[
            pl.BlockSpec((BT, 1), lambda i: (i, 0)),
            pl.BlockSpec((BT, DEG, None), lambda i: (i, 0, 0)),
            pl.BlockSpec((BT, DEG, D), lambda i: (i, 0, 0)),
        ],
        out_specs=pl.BlockSpec((BT, D), lambda i: (i, 0)),
        out_shape=jax.ShapeDtypeStruct((NTC, D), jnp.float32),
        compiler_params=pltpu.CompilerParams(
            dimension_semantics=("arbitrary",)),
    )(self_a, attn, feat)


_BCAST_DN = lax.GatherDimensionNumbers(
    offset_dims=(), collapsed_slice_dims=(0,), start_index_map=(0,))


def _bcast(vec, n):
    idx = jnp.full((LANES, 1), n, jnp.int32)
    return lax.gather(vec, idx, _BCAST_DN, slice_sizes=(1,),
                      mode=lax.GatherScatterMode.PROMISE_IN_BOUNDS)


def _compute_block(i, slot, oslot, sa_chunk, attn_chunk, feat_bufs, out_bufs):
    i16 = lax.iota(jnp.int32, LANES)
    z16 = jnp.zeros((LANES,), jnp.int32)
    node0 = i * NB
    sa = plsc.load_gather(sa_chunk, [node0 + i16, z16])
    for n in range(NB):
        nrow = node0 + n
        arow = nrow * DEG
        a0 = plsc.load_gather(attn_chunk, [arow + i16])
        a1 = plsc.load_gather(attn_chunk, [arow + LANES + i16])
        sn = _bcast(sa, n)
        a0 = a0 + sn
        a1 = a1 + sn
        l0 = jnp.maximum(a0, a0 * SLOPE)
        l1 = jnp.maximum(a1, a1 * SLOPE)
        m = jnp.max(jnp.maximum(l0, l1))
        e0 = jnp.exp(l0 - m)
        e1 = jnp.exp(l1 - m)
        s = jnp.sum(e0) + jnp.sum(e1)
        rv = 1.0 / jnp.broadcast_to(s, (LANES,))
        w0 = e0 * rv
        w1 = e1 * rv
        acc = [jnp.zeros((LANES,), jnp.float32)] * NJ
        for k in range(DEG):
            w = _bcast(w0 if k < LANES else w1, k % LANES)
            for j in range(NJ):
                acc[j] = acc[j] + w * feat_bufs[slot, n, k,
                                                pl.ds(j * LANES, LANES)]
        for j in range(NJ):
            out_bufs[oslot, n, pl.ds(j * LANES, LANES)] = acc[j]


def _sc_body(self_a, attn, feat, out, sa_chunk, attn_chunk, feat_bufs,
             out_bufs, feat_sems, out_sems):
    wid = lax.axis_index("c") * 16 + lax.axis_index("s")
    start_blk = (wid * NBLK) // NW
    end_blk = ((wid + 1) * NBLK) // NW
    nblk = end_blk - start_blk
    gbase = NTC + start_blk * NB

    pltpu.sync_copy(self_a.at[pl.ds(gbase, MAXB * NB)],
                    sa_chunk.at[pl.ds(0, MAXB * NB)])
    pltpu.sync_copy(attn.at[pl.ds(gbase * DEG, MAXB * NB * DEG)], attn_chunk)

    for b in range(RING):
        pltpu.async_copy(feat.at[pl.ds(gbase + b * NB, NB)],
                         feat_bufs.at[b], feat_sems.at[b])

    def body(i, carry):
        slot = lax.rem(i, RING)
        oslot = lax.rem(i, 2)
        gnode = gbase + i * NB
        lnode = gnode - NTC
        pltpu.make_async_copy(feat.at[pl.ds(gnode, NB)],
                              feat_bufs.at[slot], feat_sems.at[slot]).wait()

        @pl.when(i >= 2)
        def _():
            pltpu.make_async_copy(out_bufs.at[oslot],
                                  out.at[pl.ds(lnode - 2 * NB, NB)],
                                  out_sems.at[oslot]).wait()

        _compute_block(i, slot, oslot, sa_chunk, attn_chunk, feat_bufs,
                       out_bufs)
        pltpu.async_copy(out_bufs.at[oslot], out.at[pl.ds(lnode, NB)],
                         out_sems.at[oslot])

        @pl.when(i + RING < nblk)
        def _():
            pltpu.async_copy(feat.at[pl.ds(gnode + RING * NB, NB)],
                             feat_bufs.at[slot], feat_sems.at[slot])

        return carry

    lax.fori_loop(0, nblk, body, 0)

    for d in (2, 1):
        i = nblk - d
        oslot = lax.rem(i, 2)
        pltpu.make_async_copy(out_bufs.at[oslot],
                              out.at[pl.ds((start_blk + i) * NB, NB)],
                              out_sems.at[oslot]).wait()


def _sc_call(self_a, at1, feat):
    mesh = plsc.VectorSubcoreMesh(core_axis_name="c", subcore_axis_name="s")
    f = pl.kernel(
        _sc_body,
        out_type=jax.ShapeDtypeStruct((NSC, D), jnp.float32),
        mesh=mesh,
        scratch_types=[
            pltpu.VMEM((MAXB * NB + LANES, 1), jnp.float32),
            pltpu.VMEM((MAXB * NB * DEG,), jnp.float32),
            pltpu.VMEM((RING, NB, DEG, D), jnp.float32),
            pltpu.VMEM((2, NB, D), jnp.float32),
            pltpu.SemaphoreType.DMA((RING,)),
            pltpu.SemaphoreType.DMA((2,)),
        ],
        compiler_params=pltpu.CompilerParams(needs_layout_passes=False),
    )
    return f(self_a, at1, feat)


def kernel(self_a, attn, feat):
    at1 = attn.reshape(N * DEG)
    o_sc = _sc_call(self_a, at1, feat)
    o_tc = _tc_call(self_a, attn, feat)
    return jnp.concatenate([o_tc, o_sc], axis=0)


# speedup vs baseline: 1.4462x; 1.4462x over previous
import jax
import jax.numpy as jnp
from jax import lax
from jax.experimental import pallas as pl
from jax.experimental.pallas import tpu as pltpu
from jax.experimental.pallas import tpu_sc as plsc

N = 10000
DEG = 32
D = 128
SLOPE = 0.01

BT = 1024
NTC = 7168
NSC = N - NTC

NB = 8
NBLK = NSC // NB
NW = 32
MAXB = -(-NBLK // NW)
LANES = 16
NJ = D // LANES
RING = 2
FB = NB * DEG * D


def _tc_body(sa_ref, attn_ref, feat_ref, out_ref):
    a = attn_ref[...] + sa_ref[...]
    l = jnp.maximum(a, a * SLOPE)
    m = jnp.max(l, axis=1, keepdims=True)
    e = jnp.exp(l - m)
    w = e / jnp.sum(e, axis=1, keepdims=True)
    acc = w[:, 0:1] * feat_ref[:, 0, :]
    for k in range(1, DEG):
        acc = acc + w[:, k:k + 1] * feat_ref[:, k, :]
    out_ref[...] = acc


def _tc_call(self_a, attn, feat):
    return pl.pallas_call(
        _tc_body,
        grid=(NTC // BT,),
        in_specs=[
            pl.BlockSpec((BT, 1), lambda i: (i, 0)),
            pl.BlockSpec((BT, DEG), lambda i: (i, 0)),
            pl.BlockSpec((BT, DEG, D), lambda i: (i, 0, 0)),
        ],
        out_specs=pl.BlockSpec((BT, D), lambda i: (i, 0)),
        out_shape=jax.ShapeDtypeStruct((NTC, D), jnp.float32),
        compiler_params=pltpu.CompilerParams(
            dimension_semantics=("arbitrary",)),
    )(self_a, attn, feat)


_BCAST_DN = lax.GatherDimensionNumbers(
    offset_dims=(), collapsed_slice_dims=(0,), start_index_map=(0,))


def _bcast(vec, n):
    idx = jnp.full((LANES, 1), n, jnp.int32)
    return lax.gather(vec, idx, _BCAST_DN, slice_sizes=(1,),
                      mode=lax.GatherScatterMode.PROMISE_IN_BOUNDS)


def _compute_block(i, slot, oslot, sa_chunk, attn_chunk, feat_bufs, out_bufs):
    i16 = lax.iota(jnp.int32, LANES)
    z16 = jnp.zeros((LANES,), jnp.int32)
    node0 = i * NB
    sa = plsc.load_gather(sa_chunk, [node0 + i16, z16])
    for n in range(NB):
        nrow = node0 + n
        nsplat = jnp.full((LANES,), nrow, jnp.int32)
        a0 = plsc.load_gather(attn_chunk, [nsplat, i16])
        a1 = plsc.load_gather(attn_chunk, [nsplat, LANES + i16])
        sn = _bcast(sa, n)
        a0 = a0 + sn
        a1 = a1 + sn
        l0 = jnp.maximum(a0, a0 * SLOPE)
        l1 = jnp.maximum(a1, a1 * SLOPE)
        m = jnp.max(jnp.maximum(l0, l1))
        e0 = jnp.exp(l0 - m)
        e1 = jnp.exp(l1 - m)
        s = jnp.sum(e0) + jnp.sum(e1)
        rv = 1.0 / jnp.broadcast_to(s, (LANES,))
        w0 = e0 * rv
        w1 = e1 * rv
        acc = [jnp.zeros((LANES,), jnp.float32)] * NJ
        for k in range(DEG):
            w = _bcast(w0 if k < LANES else w1, k % LANES)
            for j in range(NJ):
                acc[j] = acc[j] + w * feat_bufs[slot, n, k,
                                                pl.ds(j * LANES, LANES)]
        for j in range(NJ):
            out_bufs[oslot, n, pl.ds(j * LANES, LANES)] = acc[j]


def _sc_body(self_a, attn, feat, out, sa_chunk, attn_chunk, feat_bufs,
             out_bufs, feat_sems, out_sems):
    wid = lax.axis_index("c") * 16 + lax.axis_index("s")
    start_blk = (wid * NBLK) // NW
    end_blk = ((wid + 1) * NBLK) // NW
    nblk = end_blk - start_blk
    gbase = NTC + start_blk * NB

    pltpu.sync_copy(self_a.at[pl.ds(gbase, MAXB * NB)],
                    sa_chunk.at[pl.ds(0, MAXB * NB)])
    pltpu.sync_copy(attn.at[pl.ds(gbase, MAXB * NB)], attn_chunk)

    for b in range(RING):
        pltpu.async_copy(feat.at[pl.ds(gbase + b * NB, NB)],
                         feat_bufs.at[b], feat_sems.at[b])

    def body(i, carry):
        slot = lax.rem(i, RING)
        oslot = lax.rem(i, 2)
        gnode = gbase + i * NB
        lnode = gnode - NTC
        pltpu.make_async_copy(feat.at[pl.ds(gnode, NB)],
                              feat_bufs.at[slot], feat_sems.at[slot]).wait()

        @pl.when(i >= 2)
        def _():
            pltpu.make_async_copy(out_bufs.at[oslot],
                                  out.at[pl.ds(lnode - 2 * NB, NB)],
                                  out_sems.at[oslot]).wait()

        _compute_block(i, slot, oslot, sa_chunk, attn_chunk, feat_bufs,
                       out_bufs)
        pltpu.async_copy(out_bufs.at[oslot], out.at[pl.ds(lnode, NB)],
                         out_sems.at[oslot])

        @pl.when(i + RING < nblk)
        def _():
            pltpu.async_copy(feat.at[pl.ds(gnode + RING * NB, NB)],
                             feat_bufs.at[slot], feat_sems.at[slot])

        return carry

    lax.fori_loop(0, nblk, body, 0)

    for d in (2, 1):
        i = nblk - d
        oslot = lax.rem(i, 2)
        pltpu.make_async_copy(out_bufs.at[oslot],
                              out.at[pl.ds((start_blk + i) * NB, NB)],
                              out_sems.at[oslot]).wait()


def _sc_call(self_a, at1, feat):
    mesh = plsc.VectorSubcoreMesh(core_axis_name="c", subcore_axis_name="s")
    f = pl.kernel(
        _sc_body,
        out_type=jax.ShapeDtypeStruct((NSC, D), jnp.float32),
        mesh=mesh,
        scratch_types=[
            pltpu.VMEM((MAXB * NB + LANES, 1), jnp.float32),
            pltpu.VMEM((MAXB * NB, DEG), jnp.float32),
            pltpu.VMEM((RING, NB, DEG, D), jnp.float32),
            pltpu.VMEM((2, NB, D), jnp.float32),
            pltpu.SemaphoreType.DMA((RING,)),
            pltpu.SemaphoreType.DMA((2,)),
        ],
        compiler_params=pltpu.CompilerParams(needs_layout_passes=False),
    )
    return f(self_a, at1, feat)


def kernel(self_a, attn, feat):
    at2 = attn.reshape(N, DEG)
    o_sc = _sc_call(self_a, at2, feat)
    o_tc = _tc_call(self_a, at2, feat)
    return jnp.concatenate([o_tc, o_sc], axis=0)


# speedup vs baseline: 2.3647x; 1.6351x over previous
import jax
import jax.numpy as jnp
from jax import lax
from jax.experimental import pallas as pl
from jax.experimental.pallas import tpu as pltpu
from jax.experimental.pallas import tpu_sc as plsc

N = 10000
DEG = 32
D = 128
SLOPE = 0.01

BT = 512
NTC = 7168
NSC = N - NTC

NB = 8
NBLK = NSC // NB
NW = 32
MAXB = -(-NBLK // NW)
LANES = 16
NJ = D // LANES
RING = 3


def _tc_body(sa_ref, attn_ref, feat_ref, out_ref):
    a = attn_ref[...] + sa_ref[...]
    l = jnp.maximum(a, a * SLOPE)
    m = jnp.max(l, axis=1, keepdims=True)
    e = jnp.exp(l - m)
    w = e / jnp.sum(e, axis=1, keepdims=True)
    out_ref[...] = jnp.sum(w[:, :, None] * feat_ref[...], axis=1)


def _tc_call(self_a, at2, feat):
    return pl.pallas_call(
        _tc_body,
        grid=(NTC // BT,),
        in_specs=[
            pl.BlockSpec((BT, 1), lambda i: (i, 0)),
            pl.BlockSpec((BT, DEG), lambda i: (i, 0)),
            pl.BlockSpec((BT, DEG, D), lambda i: (i, 0, 0)),
        ],
        out_specs=pl.BlockSpec((BT, D), lambda i: (i, 0)),
        out_shape=jax.ShapeDtypeStruct((NTC, D), jnp.float32),
        compiler_params=pltpu.CompilerParams(
            dimension_semantics=("arbitrary",)),
    )(self_a, at2, feat)


_BCAST_DN = lax.GatherDimensionNumbers(
    offset_dims=(), collapsed_slice_dims=(0,), start_index_map=(0,))


def _bcast(vec, n):
    idx = jnp.full((LANES, 1), n, jnp.int32)
    return lax.gather(vec, idx, _BCAST_DN, slice_sizes=(1,),
                      mode=lax.GatherScatterMode.PROMISE_IN_BOUNDS)


def _compute_block(i, slot, oslot, sa_chunk, attn_chunk, feat_bufs, out_bufs):
    i16 = lax.iota(jnp.int32, LANES)
    z16 = jnp.zeros((LANES,), jnp.int32)
    node0 = i * NB
    sa = plsc.load_gather(sa_chunk, [node0 + i16, z16])
    for n in range(NB):
        arow = (node0 + n) * DEG
        a0 = plsc.load_gather(attn_chunk, [arow + i16])
        a1 = plsc.load_gather(attn_chunk, [arow + LANES + i16])
        sn = _bcast(sa, n)
        a0 = a0 + sn
        a1 = a1 + sn
        l0 = jnp.maximum(a0, a0 * SLOPE)
        l1 = jnp.maximum(a1, a1 * SLOPE)
        m = jnp.max(jnp.maximum(l0, l1))
        e0 = jnp.exp(l0 - m)
        e1 = jnp.exp(l1 - m)
        s = jnp.sum(e0) + jnp.sum(e1)
        rv = 1.0 / jnp.broadcast_to(s, (LANES,))
        w0 = e0 * rv
        w1 = e1 * rv
        acc = [jnp.zeros((LANES,), jnp.float32)] * NJ
        for k in range(DEG):
            w = _bcast(w0 if k < LANES else w1, k % LANES)
            for j in range(NJ):
                acc[j] = acc[j] + w * feat_bufs[slot, n, k,
                                                pl.ds(j * LANES, LANES)]
        for j in range(NJ):
            out_bufs[oslot, n, pl.ds(j * LANES, LANES)] = acc[j]


def _sc_body(self_a, at1, feat, out, sa_chunk, attn_chunk, feat_bufs,
             out_bufs, feat_sems, out_sems):
    wid = lax.axis_index("c") * 16 + lax.axis_index("s")
    start_blk = (wid * NBLK) // NW
    end_blk = ((wid + 1) * NBLK) // NW
    nblk = end_blk - start_blk
    gbase = NTC + start_blk * NB

    pltpu.sync_copy(self_a.at[pl.ds(gbase, MAXB * NB)],
                    sa_chunk.at[pl.ds(0, MAXB * NB)])
    pltpu.sync_copy(at1.at[pl.ds(gbase * DEG, MAXB * NB * DEG)], attn_chunk)

    for b in range(RING):
        pltpu.async_copy(feat.at[pl.ds(gbase + b * NB, NB)],
                         feat_bufs.at[b], feat_sems.at[b])

    def body(i, carry):
        slot = lax.rem(i, RING)
        oslot = lax.rem(i, 2)
        gnode = gbase + i * NB
        lnode = gnode - NTC
        pltpu.make_async_copy(feat.at[pl.ds(gnode, NB)],
                              feat_bufs.at[slot], feat_sems.at[slot]).wait()

        @pl.when(i >= 2)
        def _():
            pltpu.make_async_copy(out_bufs.at[oslot],
                                  out.at[pl.ds(lnode - 2 * NB, NB)],
                                  out_sems.at[oslot]).wait()

        _compute_block(i, slot, oslot, sa_chunk, attn_chunk, feat_bufs,
                       out_bufs)
        pltpu.async_copy(out_bufs.at[oslot], out.at[pl.ds(lnode, NB)],
                         out_sems.at[oslot])

        @pl.when(i + RING < nblk)
        def _():
            pltpu.async_copy(feat.at[pl.ds(gnode + RING * NB, NB)],
                             feat_bufs.at[slot], feat_sems.at[slot])

        return carry

    lax.fori_loop(0, nblk, body, 0)

    for d in (2, 1):
        i = nblk - d
        oslot = lax.rem(i, 2)
        pltpu.make_async_copy(out_bufs.at[oslot],
                              out.at[pl.ds((start_blk + i) * NB, NB)],
                              out_sems.at[oslot]).wait()


def _sc_call(self_a, at1, feat):
    mesh = plsc.VectorSubcoreMesh(core_axis_name="c", subcore_axis_name="s")
    f = pl.kernel(
        _sc_body,
        out_type=jax.ShapeDtypeStruct((NSC, D), jnp.float32),
        mesh=mesh,
        scratch_types=[
            pltpu.VMEM((MAXB * NB + LANES, 1), jnp.float32),
            pltpu.VMEM((MAXB * NB * DEG,), jnp.float32),
            pltpu.VMEM((RING, NB, DEG, D), jnp.float32),
            pltpu.VMEM((2, NB, D), jnp.float32),
            pltpu.SemaphoreType.DMA((RING,)),
            pltpu.SemaphoreType.DMA((2,)),
        ],
        compiler_params=pltpu.CompilerParams(needs_layout_passes=False),
    )
    return f(self_a, at1, feat)


def kernel(self_a, attn, feat):
    at1 = attn.reshape(N * DEG)
    at2 = attn.reshape(N, DEG)
    o_sc = _sc_call(self_a, at1, feat)
    o_tc = _tc_call(self_a, at2, feat)
    return jnp.concatenate([o_tc, o_sc], axis=0)


# speedup vs baseline: 2.4274x; 1.0265x over previous
import jax
import jax.numpy as jnp
from jax import lax
from jax.experimental import pallas as pl
from jax.experimental.pallas import tpu as pltpu
from jax.experimental.pallas import tpu_sc as plsc

N = 10000
DEG = 32
D = 128
SLOPE = 0.01

BT = 512
NTC = 7168
NSC = N - NTC

NB = 8
NBLK = NSC // NB
NW = 32
MAXB = -(-NBLK // NW)
LANES = 16
NJ = D // LANES
RING = 2


def _tc_body(sa_ref, attn_ref, feat_ref, out_ref):
    a = attn_ref[...] + sa_ref[...]
    l = jnp.maximum(a, a * SLOPE)
    m = jnp.max(l, axis=1, keepdims=True)
    e = jnp.exp(l - m)
    w = e / jnp.sum(e, axis=1, keepdims=True)
    out_ref[...] = jnp.sum(w[:, :, None] * feat_ref[...], axis=1)


def _tc_call(self_a, at2, feat):
    return pl.pallas_call(
        _tc_body,
        grid=(NTC // BT,),
        in_specs=[
            pl.BlockSpec((BT, 1), lambda i: (i, 0)),
            pl.BlockSpec((BT, DEG), lambda i: (i, 0)),
            pl.BlockSpec((BT, DEG, D), lambda i: (i, 0, 0)),
        ],
        out_specs=pl.BlockSpec((BT, D), lambda i: (i, 0)),
        out_shape=jax.ShapeDtypeStruct((NTC, D), jnp.float32),
        compiler_params=pltpu.CompilerParams(
            dimension_semantics=("arbitrary",)),
    )(self_a, at2, feat)


_BCAST_DN = lax.GatherDimensionNumbers(
    offset_dims=(), collapsed_slice_dims=(0,), start_index_map=(0,))


def _bcast(vec, n):
    idx = jnp.full((LANES, 1), n, jnp.int32)
    return lax.gather(vec, idx, _BCAST_DN, slice_sizes=(1,),
                      mode=lax.GatherScatterMode.PROMISE_IN_BOUNDS)


def _compute_block(i, slot, oslot, sa_chunk, attn_chunk, feat_bufs, out_bufs):
    i16 = lax.iota(jnp.int32, LANES)
    z16 = jnp.zeros((LANES,), jnp.int32)
    node0 = i * NB
    sa = plsc.load_gather(sa_chunk, [node0 + i16, z16])
    for n in range(NB):
        nsplat = jnp.full((LANES,), node0 + n, jnp.int32)
        a0 = plsc.load_gather(attn_chunk, [nsplat, i16])
        a1 = plsc.load_gather(attn_chunk, [nsplat, LANES + i16])
        sn = _bcast(sa, n)
        a0 = a0 + sn
        a1 = a1 + sn
        l0 = jnp.maximum(a0, a0 * SLOPE)
        l1 = jnp.maximum(a1, a1 * SLOPE)
        m = jnp.max(jnp.maximum(l0, l1))
        e0 = jnp.exp(l0 - m)
        e1 = jnp.exp(l1 - m)
        s = jnp.sum(e0) + jnp.sum(e1)
        rv = 1.0 / jnp.broadcast_to(s, (LANES,))
        w0 = e0 * rv
        w1 = e1 * rv
        acc = [jnp.zeros((LANES,), jnp.float32)] * NJ
        for k in range(DEG):
            w = _bcast(w0 if k < LANES else w1, k % LANES)
            for j in range(NJ):
                acc[j] = acc[j] + w * feat_bufs[slot, n, k,
                                                pl.ds(j * LANES, LANES)]
        for j in range(NJ):
            out_bufs[oslot, n, pl.ds(j * LANES, LANES)] = acc[j]


def _sc_body(self_a, at1, feat, out, sa_chunk, attn_chunk, feat_bufs,
             out_bufs, feat_sems, out_sems):
    wid = lax.axis_index("c") * 16 + lax.axis_index("s")
    start_blk = (wid * NBLK) // NW
    end_blk = ((wid + 1) * NBLK) // NW
    nblk = end_blk - start_blk
    gbase = NTC + start_blk * NB

    pltpu.sync_copy(self_a.at[pl.ds(gbase, MAXB * NB)],
                    sa_chunk.at[pl.ds(0, MAXB * NB)])
    pltpu.sync_copy(at1.at[pl.ds(gbase, MAXB * NB)], attn_chunk)

    for b in range(RING):
        pltpu.async_copy(feat.at[pl.ds(gbase + b * NB, NB)],
                         feat_bufs.at[b], feat_sems.at[b])

    def body(i, carry):
        slot = lax.rem(i, RING)
        oslot = lax.rem(i, 2)
        gnode = gbase + i * NB
        lnode = gnode - NTC
        pltpu.make_async_copy(feat.at[pl.ds(gnode, NB)],
                              feat_bufs.at[slot], feat_sems.at[slot]).wait()

        @pl.when(i >= 2)
        def _():
            pltpu.make_async_copy(out_bufs.at[oslot],
                                  out.at[pl.ds(lnode - 2 * NB, NB)],
                                  out_sems.at[oslot]).wait()

        _compute_block(i, slot, oslot, sa_chunk, attn_chunk, feat_bufs,
                       out_bufs)
        pltpu.async_copy(out_bufs.at[oslot], out.at[pl.ds(lnode, NB)],
                         out_sems.at[oslot])

        @pl.when(i + RING < nblk)
        def _():
            pltpu.async_copy(feat.at[pl.ds(gnode + RING * NB, NB)],
                             feat_bufs.at[slot], feat_sems.at[slot])

        return carry

    lax.fori_loop(0, nblk, body, 0)

    for d in (2, 1):
        i = nblk - d
        oslot = lax.rem(i, 2)
        pltpu.make_async_copy(out_bufs.at[oslot],
                              out.at[pl.ds((start_blk + i) * NB, NB)],
                              out_sems.at[oslot]).wait()


def _sc_call(self_a, at1, feat):
    mesh = plsc.VectorSubcoreMesh(core_axis_name="c", subcore_axis_name="s")
    f = pl.kernel(
        _sc_body,
        out_type=jax.ShapeDtypeStruct((NSC, D), jnp.float32),
        mesh=mesh,
        scratch_types=[
            pltpu.VMEM((MAXB * NB + LANES, 1), jnp.float32),
            pltpu.VMEM((MAXB * NB, DEG), jnp.float32),
            pltpu.VMEM((RING, NB, DEG, D), jnp.float32),
            pltpu.VMEM((2, NB, D), jnp.float32),
            pltpu.SemaphoreType.DMA((RING,)),
            pltpu.SemaphoreType.DMA((2,)),
        ],
        compiler_params=pltpu.CompilerParams(needs_layout_passes=False),
    )
    return f(self_a, at1, feat)


def kernel(self_a, attn, feat):
    at2 = attn.reshape(N, DEG)
    o_sc = _sc_call(self_a, at2, feat)
    o_tc = _tc_call(self_a, at2, feat)
    return jnp.concatenate([o_tc, o_sc], axis=0)


# speedup vs baseline: 2.4526x; 1.0104x over previous
import jax
import jax.numpy as jnp
from jax import lax
from jax.experimental import pallas as pl
from jax.experimental.pallas import tpu as pltpu
from jax.experimental.pallas import tpu_sc as plsc

N = 10000
DEG = 32
D = 128
SLOPE = 0.01

BT = 512
NTC = 7168
NSC = N - NTC

NB = 8
NBLK = NSC // NB
NW = 32
MAXB = -(-NBLK // NW)
LANES = 16
NJ = D // LANES
RING = 2


def _tc_body(sa_ref, attn_ref, feat_ref, out_ref):
    a = attn_ref[...] + sa_ref[...]
    l = jnp.maximum(a, a * SLOPE)
    m = jnp.max(l, axis=1, keepdims=True)
    e = jnp.exp(l - m)
    w = e / jnp.sum(e, axis=1, keepdims=True)
    out_ref[...] = jnp.sum(w[:, :, None] * feat_ref[...], axis=1)


def _tc_call(self_a, at2, feat):
    return pl.pallas_call(
        _tc_body,
        grid=(NTC // BT,),
        in_specs=[
            pl.BlockSpec((BT, 1), lambda i: (i, 0)),
            pl.BlockSpec((BT, DEG), lambda i: (i, 0)),
            pl.BlockSpec((BT, DEG, D), lambda i: (i, 0, 0)),
        ],
        out_specs=pl.BlockSpec((BT, D), lambda i: (i, 0)),
        out_shape=jax.ShapeDtypeStruct((NTC, D), jnp.float32),
        compiler_params=pltpu.CompilerParams(
            dimension_semantics=("parallel",)),
    )(self_a, at2, feat)


_BCAST_DN = lax.GatherDimensionNumbers(
    offset_dims=(), collapsed_slice_dims=(0,), start_index_map=(0,))


def _bcast(vec, n):
    idx = jnp.full((LANES, 1), n, jnp.int32)
    return lax.gather(vec, idx, _BCAST_DN, slice_sizes=(1,),
                      mode=lax.GatherScatterMode.PROMISE_IN_BOUNDS)


def _compute_block(i, slot, oslot, sa_chunk, attn_chunk, feat_bufs, out_bufs):
    i16 = lax.iota(jnp.int32, LANES)
    z16 = jnp.zeros((LANES,), jnp.int32)
    node0 = i * NB
    sa = plsc.load_gather(sa_chunk, [node0 + i16, z16])
    for n in range(NB):
        nsplat = jnp.full((LANES,), node0 + n, jnp.int32)
        a0 = plsc.load_gather(attn_chunk, [nsplat, i16])
        a1 = plsc.load_gather(attn_chunk, [nsplat, LANES + i16])
        sn = _bcast(sa, n)
        a0 = a0 + sn
        a1 = a1 + sn
        l0 = jnp.maximum(a0, a0 * SLOPE)
        l1 = jnp.maximum(a1, a1 * SLOPE)
        m = jnp.max(jnp.maximum(l0, l1))
        e0 = jnp.exp(l0 - m)
        e1 = jnp.exp(l1 - m)
        s = jnp.sum(e0) + jnp.sum(e1)
        rv = 1.0 / jnp.broadcast_to(s, (LANES,))
        w0 = e0 * rv
        w1 = e1 * rv
        acc = [jnp.zeros((LANES,), jnp.float32)] * NJ
        for k in range(DEG):
            w = _bcast(w0 if k < LANES else w1, k % LANES)
            for j in range(NJ):
                acc[j] = acc[j] + w * feat_bufs[slot, n, k,
                                                pl.ds(j * LANES, LANES)]
        for j in range(NJ):
            out_bufs[oslot, n, pl.ds(j * LANES, LANES)] = acc[j]


def _sc_body(self_a, at1, feat, out, sa_chunk, attn_chunk, feat_bufs,
             out_bufs, feat_sems, out_sems):
    wid = lax.axis_index("c") * 16 + lax.axis_index("s")
    start_blk = (wid * NBLK) // NW
    end_blk = ((wid + 1) * NBLK) // NW
    nblk = end_blk - start_blk
    gbase = NTC + start_blk * NB

    pltpu.sync_copy(self_a.at[pl.ds(gbase, MAXB * NB)],
                    sa_chunk.at[pl.ds(0, MAXB * NB)])
    pltpu.sync_copy(at1.at[pl.ds(gbase, MAXB * NB)], attn_chunk)

    for b in range(RING):
        pltpu.async_copy(feat.at[pl.ds(gbase + b * NB, NB)],
                         feat_bufs.at[b], feat_sems.at[b])

    def body(i, carry):
        slot = lax.rem(i, RING)
        oslot = lax.rem(i, 2)
        gnode = gbase + i * NB
        lnode = gnode - NTC
        pltpu.make_async_copy(feat.at[pl.ds(gnode, NB)],
                              feat_bufs.at[slot], feat_sems.at[slot]).wait()

        @pl.when(i >= 2)
        def _():
            pltpu.make_async_copy(out_bufs.at[oslot],
                                  out.at[pl.ds(lnode - 2 * NB, NB)],
                                  out_sems.at[oslot]).wait()

        _compute_block(i, slot, oslot, sa_chunk, attn_chunk, feat_bufs,
                       out_bufs)
        pltpu.async_copy(out_bufs.at[oslot], out.at[pl.ds(lnode, NB)],
                         out_sems.at[oslot])

        @pl.when(i + RING < nblk)
        def _():
            pltpu.async_copy(feat.at[pl.ds(gnode + RING * NB, NB)],
                             feat_bufs.at[slot], feat_sems.at[slot])

        return carry

    lax.fori_loop(0, nblk, body, 0)

    for d in (2, 1):
        i = nblk - d
        oslot = lax.rem(i, 2)
        pltpu.make_async_copy(out_bufs.at[oslot],
                              out.at[pl.ds((start_blk + i) * NB, NB)],
                              out_sems.at[oslot]).wait()


def _sc_call(self_a, at1, feat):
    mesh = plsc.VectorSubcoreMesh(core_axis_name="c", subcore_axis_name="s")
    f = pl.kernel(
        _sc_body,
        out_type=jax.ShapeDtypeStruct((NSC, D), jnp.float32),
        mesh=mesh,
        scratch_types=[
            pltpu.VMEM((MAXB * NB + LANES, 1), jnp.float32),
            pltpu.VMEM((MAXB * NB, DEG), jnp.float32),
            pltpu.VMEM((RING, NB, DEG, D), jnp.float32),
            pltpu.VMEM((2, NB, D), jnp.float32),
            pltpu.SemaphoreType.DMA((RING,)),
            pltpu.SemaphoreType.DMA((2,)),
        ],
        compiler_params=pltpu.CompilerParams(needs_layout_passes=False),
    )
    return f(self_a, at1, feat)


def kernel(self_a, attn, feat):
    at2 = attn.reshape(N, DEG)
    o_sc = _sc_call(self_a, at2, feat)
    o_tc = _tc_call(self_a, at2, feat)
    return jnp.concatenate([o_tc, o_sc], axis=0)
